# Initial kernel scaffold; baseline (speedup 1.0000x reference)
import jax
import jax.numpy as jnp
from jax.experimental import pallas as pl


def kernel(x, gate_w, w1, w2, w3, sw1, sw2, sw3):
    raise NotImplementedError("write your pallas kernel here")


# speedup vs baseline: 2.6783x; 2.6783x over previous
import functools

import jax
import jax.numpy as jnp
from jax.experimental import pallas as pl
from jax.experimental.pallas import tpu as pltpu

DIM = 768
INTER = 256
E = 8
TOPK = 2
SI = 512
T = 2048


def _moe_kernel(x_ref, gw_ref, w1_ref, w2_ref, w3_ref, sw1_ref, sw2_ref, sw3_ref,
                out_ref, combine_ref, xb_ref):
    e = pl.program_id(0)

    @pl.when(e == 0)
    def _init():
        xf = x_ref[...]
        xb = xf.astype(jnp.bfloat16)
        xb_ref[...] = xb
        scores = jax.nn.sigmoid(
            jax.lax.dot_general(xf, gw_ref[...], (((1,), (1,)), ((), ())),
                                preferred_element_type=jnp.float32))
        m1 = jnp.max(scores, axis=1, keepdims=True)
        i1 = jnp.argmax(scores, axis=1)[:, None]
        eids = jax.lax.broadcasted_iota(jnp.int32, (T, E), 1)
        masked = jnp.where(eids == i1, -jnp.inf, scores)
        m2 = jnp.max(masked, axis=1, keepdims=True)
        i2 = jnp.argmax(masked, axis=1)[:, None]
        denom = m1 + m2
        c1 = m1 / denom
        c2 = m2 / denom
        combine_ref[...] = (jnp.where(eids == i1, c1, 0.0)
                            + jnp.where(eids == i2, c2, 0.0))
        a = jax.lax.dot(xb, sw1_ref[...], preferred_element_type=jnp.float32)
        b = jax.lax.dot(xb, sw3_ref[...], preferred_element_type=jnp.float32)
        hs = (jax.nn.silu(a) * b).astype(jnp.bfloat16)
        out_ref[...] = jax.lax.dot(hs, sw2_ref[...],
                                   preferred_element_type=jnp.float32)

    xb = xb_ref[...]
    a = jax.lax.dot(xb, w1_ref[0], preferred_element_type=jnp.float32)
    b = jax.lax.dot(xb, w3_ref[0], preferred_element_type=jnp.float32)
    h = (jax.nn.silu(a) * b).astype(jnp.bfloat16)
    y = jax.lax.dot(h, w2_ref[0], preferred_element_type=jnp.float32)
    cmb = combine_ref[...]
    lane = jax.lax.broadcasted_iota(jnp.int32, (T, E), 1)
    ce = jnp.sum(jnp.where(lane == e, cmb, 0.0), axis=1, keepdims=True)
    out_ref[...] += y * ce


@jax.jit
def kernel(x, gate_w, w1, w2, w3, sw1, sw2, sw3):
    shape = x.shape
    xt = x.reshape(-1, DIM)
    w1b = w1.astype(jnp.bfloat16)
    w2b = w2.astype(jnp.bfloat16)
    w3b = w3.astype(jnp.bfloat16)
    sw1b = sw1.astype(jnp.bfloat16)
    sw2b = sw2.astype(jnp.bfloat16)
    sw3b = sw3.astype(jnp.bfloat16)

    full = lambda shp: pl.BlockSpec(shp, lambda e: (0,) * len(shp))
    per_e = lambda shp: pl.BlockSpec((1,) + shp, lambda e: (e, 0, 0))

    out = pl.pallas_call(
        _moe_kernel,
        grid=(E,),
        in_specs=[
            full((T, DIM)),
            full((E, DIM)),
            per_e((DIM, INTER)),
            per_e((INTER, DIM)),
            per_e((DIM, INTER)),
            full((DIM, SI)),
            full((SI, DIM)),
            full((DIM, SI)),
        ],
        out_specs=full((T, DIM)),
        out_shape=jax.ShapeDtypeStruct((T, DIM), jnp.float32),
        scratch_shapes=[
            pltpu.VMEM((T, E), jnp.float32),
            pltpu.VMEM((T, DIM), jnp.bfloat16),
        ],
    )(xt, gate_w, w1b, w2b, w3b, sw1b, sw2b, sw3b)
    return out.reshape(shape)
